# baseline (speedup 1.0000x reference)
import functools

import jax
import jax.numpy as jnp
from jax import lax
from jax.experimental import pallas as pl
from jax.experimental.pallas import tpu as pltpu
from jax.experimental.pallas import tpu_sc as plsc

TABLE_ROWS = 100000
TABLE_COLS = 100
BATCH = 16384

_INFO = plsc.get_sparse_core_info()
_NC = _INFO.num_cores
_NS = _INFO.num_subcores
_L = _INFO.num_lanes
_NW = _NC * _NS
_BPW = BATCH // _NW
_GCHUNK = 128

_mesh = plsc.VectorSubcoreMesh(core_axis_name="c", subcore_axis_name="s")


@functools.partial(
    pl.kernel,
    mesh=_mesh,
    out_type=jax.ShapeDtypeStruct((BATCH,), jnp.float32),
    scratch_types=[
        pltpu.VMEM((_BPW,), jnp.int32),
        pltpu.VMEM((_BPW,), jnp.int32),
        pltpu.VMEM((_BPW,), jnp.float32),
        pltpu.SemaphoreType.DMA,
    ],
)
def _gather_kernel(idx0_hbm, idx1_hbm, table_hbm, out_hbm, i0_v, i1_v, val_v, sem):
    wid = lax.axis_index("s") * _NC + lax.axis_index("c")
    base = wid * _BPW
    pltpu.sync_copy(idx0_hbm.at[pl.ds(base, _BPW)], i0_v)
    pltpu.sync_copy(idx1_hbm.at[pl.ds(base, _BPW)], i1_v)
    for i in range(_BPW // _L):
        sl = pl.ds(i * _L, _L)
        i1_v[sl] = i0_v[sl] * TABLE_COLS + i1_v[sl]
    copies = []
    for j in range(_BPW // _GCHUNK):
        sl = pl.ds(j * _GCHUNK, _GCHUNK)
        copies.append(pltpu.async_copy(table_hbm.at[i1_v.at[sl]], val_v.at[sl], sem))
    for c in copies:
        c.wait()
    pltpu.sync_copy(val_v, out_hbm.at[pl.ds(base, _BPW)])


def kernel(idx0, idx1, lookup):
    flat_table = lookup.reshape(-1)
    return _gather_kernel(idx0.astype(jnp.int32), idx1.astype(jnp.int32), flat_table)


# speedup vs baseline: 2.6203x; 2.6203x over previous
import functools

import jax
import jax.numpy as jnp
from jax import lax
from jax.experimental import pallas as pl
from jax.experimental.pallas import tpu as pltpu
from jax.experimental.pallas import tpu_sc as plsc

TABLE_ROWS = 100000
TABLE_COLS = 100
BATCH = 16384

_INFO = plsc.get_sparse_core_info()
_NC = _INFO.num_cores
_NS = _INFO.num_subcores
_L = _INFO.num_lanes
_NW = _NC * _NS
_BPW = BATCH // _NW
_G = 32
_NGRP = _BPW // _G

_mesh = plsc.VectorSubcoreMesh(core_axis_name="c", subcore_axis_name="s")


@functools.partial(
    pl.kernel,
    mesh=_mesh,
    out_type=jax.ShapeDtypeStruct((BATCH,), jnp.float32),
    scratch_types=[
        pltpu.VMEM((_BPW + _L,), jnp.int32),
        pltpu.VMEM((_BPW + _L,), jnp.int32),
        pltpu.VMEM((_G * 8, TABLE_COLS), jnp.float32),
        pltpu.VMEM((_G * 8, TABLE_COLS), jnp.float32),
        pltpu.VMEM((_BPW * _L,), jnp.float32),
        pltpu.VMEM((_BPW,), jnp.float32),
        pltpu.SemaphoreType.DMA,
        pltpu.SemaphoreType.DMA,
    ],
    compiler_params=pltpu.CompilerParams(disable_bounds_checks=True,
                                         needs_layout_passes=False),
)
def _gather_kernel(idx0_hbm, idx1_hbm, table_hbm, out_hbm,
                   i0_v, i1_v, buf0, buf1, wide_v, val_v, sem0, sem1):
    wid = lax.axis_index("s") * _NC + lax.axis_index("c")
    base = wid * _BPW
    pltpu.sync_copy(idx0_hbm.at[pl.ds(base, _BPW)], i0_v.at[pl.ds(0, _BPW)])
    pltpu.sync_copy(idx1_hbm.at[pl.ds(base, _BPW)], i1_v.at[pl.ds(0, _BPW)])

    bufs = (buf0, buf1)
    sems = (sem0, sem1)

    def fire_group(g, buf, sem):
        @plsc.parallel_loop(0, _G, step=4, unroll=1)
        def _fire(u):
            for uu in range(4):
                k = g * _G + u + uu
                r = i0_v[pl.ds(k, _L)][0]
                r8 = pl.multiple_of((r >> 3) * 8, 8)
                pltpu.async_copy(
                    table_hbm.at[pl.ds(r8, 8), :],
                    buf.at[pl.ds((u + uu) * 8, 8), :], sem)

    def drain_group(buf, sem):
        pltpu.make_async_copy(table_hbm.at[pl.ds(0, _G * 8), :], buf, sem).wait()

    def extract_group(g, buf):
        @plsc.parallel_loop(0, _G, step=4, unroll=1)
        def _extract(u):
            for uu in range(4):
                k = g * _G + u + uu
                r = i0_v[pl.ds(k, _L)][0]
                c = i1_v[pl.ds(k, _L)][0]
                row = (u + uu) * 8 + (r & 7)
                w = buf[row, pl.ds(c, _L)]
                wide_v[pl.ds(k * _L, _L)] = w

    fire_group(0, bufs[0], sems[0])
    for g in range(_NGRP):
        p = g & 1
        drain_group(bufs[p], sems[p])
        if g + 1 < _NGRP:
            fire_group(g + 1, bufs[1 - p], sems[1 - p])
        extract_group(g, bufs[p])

    @plsc.parallel_loop(0, _BPW, step=_L, unroll=1)
    def _compact(k):
        flat = (lax.iota(jnp.int32, _L) + k) * _L
        val_v[pl.ds(k, _L)] = plsc.load_gather(wide_v, [flat])

    pltpu.sync_copy(val_v, out_hbm.at[pl.ds(base, _BPW)])


def kernel(idx0, idx1, lookup):
    return _gather_kernel(idx0.astype(jnp.int32), idx1.astype(jnp.int32), lookup)


# speedup vs baseline: 2.7743x; 1.0588x over previous
import functools

import jax
import jax.numpy as jnp
from jax import lax
from jax.experimental import pallas as pl
from jax.experimental.pallas import tpu as pltpu
from jax.experimental.pallas import tpu_sc as plsc

TABLE_ROWS = 100000
TABLE_COLS = 100
BATCH = 16384

_INFO = plsc.get_sparse_core_info()
_NC = _INFO.num_cores
_NS = _INFO.num_subcores
_L = _INFO.num_lanes
_NW = _NC * _NS
_BPW = BATCH // _NW
_G = 32
_NGRP = _BPW // _G

_mesh = plsc.VectorSubcoreMesh(core_axis_name="c", subcore_axis_name="s")


@functools.partial(
    pl.kernel,
    mesh=_mesh,
    out_type=jax.ShapeDtypeStruct((BATCH,), jnp.float32),
    scratch_types=[
        pltpu.VMEM((_BPW + _L,), jnp.int32),
        pltpu.VMEM((_BPW + _L,), jnp.int32),
        pltpu.VMEM((_G * 8, TABLE_COLS), jnp.float32),
        pltpu.VMEM((_G * 8, TABLE_COLS), jnp.float32),
        pltpu.VMEM((_BPW * _L,), jnp.float32),
        pltpu.VMEM((_BPW,), jnp.float32),
        pltpu.SemaphoreType.DMA,
        pltpu.SemaphoreType.DMA,
    ],
    compiler_params=pltpu.CompilerParams(disable_bounds_checks=True,
                                         needs_layout_passes=False),
)
def _gather_kernel(idx0_hbm, idx1_hbm, table_hbm, out_hbm,
                   i0_v, i1_v, buf0, buf1, wide_v, val_v, sem0, sem1):
    wid = lax.axis_index("s") * _NC + lax.axis_index("c")
    base = wid * _BPW
    pltpu.sync_copy(idx0_hbm.at[pl.ds(base, _BPW)], i0_v.at[pl.ds(0, _BPW)])
    pltpu.sync_copy(idx1_hbm.at[pl.ds(base, _BPW)], i1_v.at[pl.ds(0, _BPW)])

    def fire_group(g, buf, sem):
        @plsc.parallel_loop(0, _G, step=4, unroll=1)
        def _fire(u):
            for uu in range(4):
                k = g * _G + u + uu
                r = i0_v[pl.ds(k, _L)][0]
                r8 = pl.multiple_of((r >> 3) * 8, 8)
                pltpu.async_copy(
                    table_hbm.at[pl.ds(r8, 8), :],
                    buf.at[pl.ds((u + uu) * 8, 8), :], sem)

    def drain_group(buf, sem):
        pltpu.make_async_copy(table_hbm.at[pl.ds(0, _G * 8), :], buf, sem).wait()

    def extract_group(g, buf):
        @plsc.parallel_loop(0, _G, step=4, unroll=1)
        def _extract(u):
            for uu in range(4):
                k = g * _G + u + uu
                r = i0_v[pl.ds(k, _L)][0]
                c = i1_v[pl.ds(k, _L)][0]
                row = (u + uu) * 8 + (r & 7)
                w = buf[row, pl.ds(c, _L)]
                wide_v[pl.ds(k * _L, _L)] = w

    fire_group(0, buf0, sem0)

    def group_body(g, carry):
        even = (g & 1) == 0

        @pl.when(even)
        def _():
            drain_group(buf0, sem0)

            @pl.when(g + 1 < _NGRP)
            def _():
                fire_group(g + 1, buf1, sem1)

            extract_group(g, buf0)

        @pl.when(jnp.logical_not(even))
        def _():
            drain_group(buf1, sem1)

            @pl.when(g + 1 < _NGRP)
            def _():
                fire_group(g + 1, buf0, sem0)

            extract_group(g, buf1)

        return carry

    lax.fori_loop(0, _NGRP, group_body, 0)

    @plsc.parallel_loop(0, _BPW, step=_L, unroll=1)
    def _compact(k):
        flat = (lax.iota(jnp.int32, _L) + k) * _L
        val_v[pl.ds(k, _L)] = plsc.load_gather(wide_v, [flat])

    pltpu.sync_copy(val_v, out_hbm.at[pl.ds(base, _BPW)])


def kernel(idx0, idx1, lookup):
    return _gather_kernel(idx0.astype(jnp.int32), idx1.astype(jnp.int32), lookup)


# speedup vs baseline: 4.6729x; 1.6843x over previous
import functools

import jax
import jax.numpy as jnp
from jax import lax
from jax.experimental import pallas as pl
from jax.experimental.pallas import tpu as pltpu
from jax.experimental.pallas import tpu_sc as plsc

TABLE_ROWS = 100000
TABLE_COLS = 100
BATCH = 16384

_INFO = plsc.get_sparse_core_info()
_NC = _INFO.num_cores
_NS = _INFO.num_subcores
_L = _INFO.num_lanes
_NW = _NC * _NS
_BPW = BATCH // _NW
_G = 32
_NGRP = _BPW // _G

_mesh = plsc.VectorSubcoreMesh(core_axis_name="c", subcore_axis_name="s")


@functools.partial(
    pl.kernel,
    mesh=_mesh,
    out_type=jax.ShapeDtypeStruct((BATCH,), jnp.float32),
    scratch_types=[
        pltpu.VMEM((_BPW + _L,), jnp.int32),
        pltpu.VMEM((_BPW + _L,), jnp.int32),
        pltpu.VMEM((_G * 8 + 1, 128), jnp.float32),
        pltpu.VMEM((_G * 8 + 1, 128), jnp.float32),
        pltpu.VMEM((_BPW * _L,), jnp.float32),
        pltpu.VMEM((_BPW,), jnp.float32),
        pltpu.SemaphoreType.DMA,
        pltpu.SemaphoreType.DMA,
    ],
    compiler_params=pltpu.CompilerParams(disable_bounds_checks=True,
                                         needs_layout_passes=False),
)
def _gather_kernel(idx0_hbm, idx1_hbm, table_t_hbm, out_hbm,
                   i0_v, i1_v, buf0, buf1, wide_v, val_v, sem0, sem1):
    wid = lax.axis_index("s") * _NC + lax.axis_index("c")
    base = wid * _BPW
    pltpu.sync_copy(idx0_hbm.at[pl.ds(base, _BPW)], i0_v.at[pl.ds(0, _BPW)])
    pltpu.sync_copy(idx1_hbm.at[pl.ds(base, _BPW)], i1_v.at[pl.ds(0, _BPW)])

    def fire_group(g, buf, sem):
        @plsc.parallel_loop(0, _G, step=4, unroll=1)
        def _fire(u):
            for uu in range(4):
                k = g * _G + u + uu
                r = i0_v[pl.ds(k, _L)][0]
                c = i1_v[pl.ds(k, _L)][0]
                c8 = pl.multiple_of((c >> 3) * 8, 8)
                r128 = pl.multiple_of((r >> 7) * 128, 128)
                pltpu.async_copy(
                    table_t_hbm.at[pl.ds(c8, 8), pl.ds(r128, 128)],
                    buf.at[pl.ds((u + uu) * 8, 8), :], sem)

    def drain_group(sem):
        for _ in range(4):
            pltpu.make_async_copy(
                out_hbm.at[pl.ds(0, _G * 256)],
                wide_v.at[pl.ds(0, _G * 256)], sem).wait()

    def extract_group(g, buf):
        @plsc.parallel_loop(0, _G, step=4, unroll=1)
        def _extract(u):
            for uu in range(4):
                k = g * _G + u + uu
                r = i0_v[pl.ds(k, _L)][0]
                c = i1_v[pl.ds(k, _L)][0]
                row = (u + uu) * 8 + (c & 7)
                w = buf[row, pl.ds(r & 127, _L)]
                wide_v[pl.ds(k * _L, _L)] = w

    fire_group(0, buf0, sem0)

    def group_body(g, carry):
        even = (g & 1) == 0

        @pl.when(even)
        def _():
            drain_group(sem0)

            @pl.when(g + 1 < _NGRP)
            def _():
                fire_group(g + 1, buf1, sem1)

            extract_group(g, buf0)

        @pl.when(jnp.logical_not(even))
        def _():
            drain_group(sem1)

            @pl.when(g + 1 < _NGRP)
            def _():
                fire_group(g + 1, buf0, sem0)

            extract_group(g, buf1)

        return carry

    lax.fori_loop(0, _NGRP, group_body, 0)

    @plsc.parallel_loop(0, _BPW, step=_L, unroll=1)
    def _compact(k):
        flat = (lax.iota(jnp.int32, _L) + k) * _L
        val_v[pl.ds(k, _L)] = plsc.load_gather(wide_v, [flat])

    pltpu.sync_copy(val_v, out_hbm.at[pl.ds(base, _BPW)])


def kernel(idx0, idx1, lookup):
    return _gather_kernel(idx0.astype(jnp.int32), idx1.astype(jnp.int32), lookup.T)
